# baseline (speedup 1.0000x reference)
import jax
import jax.numpy as jnp
from jax.experimental import pallas as pl


_TB = 256


def _add_kernel(x_ref, emb_ref, out_ref):
    out_ref[...] = x_ref[...] + emb_ref[...][None, :, :]


def kernel(x, embed_weight):
    B, T, D = x.shape
    tb = min(_TB, T)
    grid = (T // tb,)
    return pl.pallas_call(
        _add_kernel,
        grid=grid,
        in_specs=[
            pl.BlockSpec((B, tb, D), lambda i: (0, i, 0)),
            pl.BlockSpec((tb, D), lambda i: (i, 0)),
        ],
        out_specs=pl.BlockSpec((B, tb, D), lambda i: (0, i, 0)),
        out_shape=jax.ShapeDtypeStruct((B, T, D), x.dtype),
    )(x, embed_weight)


# speedup vs baseline: 1.0378x; 1.0378x over previous
import jax
import jax.numpy as jnp
from jax.experimental import pallas as pl


_TB = 2048


def _add_kernel(x_ref, emb_ref, out_ref):
    out_ref[...] = x_ref[...] + emb_ref[...][None, :, :]


def kernel(x, embed_weight):
    B, T, D = x.shape
    tb = min(_TB, T)
    grid = (T // tb, B)
    return pl.pallas_call(
        _add_kernel,
        grid=grid,
        in_specs=[
            pl.BlockSpec((1, tb, D), lambda i, b: (b, i, 0)),
            pl.BlockSpec((tb, D), lambda i, b: (i, 0)),
        ],
        out_specs=pl.BlockSpec((1, tb, D), lambda i, b: (b, i, 0)),
        out_shape=jax.ShapeDtypeStruct((B, T, D), x.dtype),
    )(x, embed_weight)
